# baseline (speedup 1.0000x reference)
import jax
import jax.numpy as jnp
from jax import lax
from jax.experimental import pallas as pl
from jax.experimental.pallas import tpu as pltpu

_K = 128
_N = 32768
_ROWS_PER_BLOCK = 16


def _rowsum(m, ones):
    return lax.dot_general(m, ones, (((1,), (0,)), ((), ())),
                           preferred_element_type=jnp.float32)


def _body(k_ref, x_ref, o_ref):
    ones = jnp.ones((_N, 1), jnp.float32)
    x = x_ref[...]
    xc = jnp.clip(x, 0.0, 1.0)
    s = _rowsum(xc, ones)
    xb = lax.bitcast_convert_type(xc, jnp.int32)

    def vstep(_, carry):
        lo, hi = carry
        mid = (lo + hi) >> 1
        cnt = _rowsum((xb >= mid).astype(jnp.float32), ones)
        ge = cnt >= _K
        return jnp.where(ge, mid, lo), jnp.where(ge, hi, mid)

    r = x.shape[0]
    lo0 = jnp.zeros((r, 1), jnp.int32)
    hi0 = jnp.full((r, 1), 0x3F800001, jnp.int32)
    lo, _ = lax.fori_loop(0, 31, vstep, (lo0, hi0))
    t = lo

    eq = xb == t
    n_ge = _rowsum((xb >= t).astype(jnp.float32), ones).astype(jnp.int32)
    n_gt = _rowsum((xb > t).astype(jnp.float32), ones).astype(jnp.int32)
    need = _K - n_gt
    idx = lax.broadcasted_iota(jnp.int32, x.shape, 1)

    def tie_bisect():
        def jstep(_, carry):
            jlo, jhi = carry
            mid = (jlo + jhi) >> 1
            c = _rowsum((eq & (idx < mid)).astype(jnp.float32), ones)
            geq = c >= need.astype(jnp.float32)
            return jnp.where(geq, jlo, mid), jnp.where(geq, mid, jhi)

        jlo0 = jnp.zeros((r, 1), jnp.int32)
        jhi0 = jnp.full((r, 1), _N, jnp.int32)
        _, jhi = lax.fori_loop(0, 16, jstep, (jlo0, jhi0))
        return jhi

    jhi = lax.cond(jnp.all(n_ge == _K),
                   lambda: jnp.full((r, 1), _N, jnp.int32),
                   tie_bisect)

    mask = (xb > t) | (eq & (idx < jhi))
    cond = s > k_ref[0, 0]
    o_ref[...] = jnp.where(cond, jnp.where(mask, xc, 0.0), xc)


def kernel(scores, k):
    rows = scores.shape[0]
    kf = jnp.asarray(k, jnp.float32).reshape(1, 1)
    grid = (rows // _ROWS_PER_BLOCK,)
    return pl.pallas_call(
        _body,
        grid=grid,
        in_specs=[
            pl.BlockSpec(memory_space=pltpu.SMEM),
            pl.BlockSpec((_ROWS_PER_BLOCK, _N), lambda i: (i, 0)),
        ],
        out_specs=pl.BlockSpec((_ROWS_PER_BLOCK, _N), lambda i: (i, 0)),
        out_shape=jax.ShapeDtypeStruct(scores.shape, scores.dtype),
    )(kf, scores)


# speedup vs baseline: 3.2351x; 3.2351x over previous
import jax
import jax.numpy as jnp
from jax import lax
from jax.experimental import pallas as pl
from jax.experimental.pallas import tpu as pltpu

_K = 128
_N = 32768
_ROWS_PER_BLOCK = 16


def _rowsum(m):
    v = m
    while v.shape[1] > 128:
        h = v.shape[1] // 2
        v = v[:, :h] + v[:, h:]
    return jnp.sum(v, axis=-1, keepdims=True)


def _count(pred):
    return _rowsum(pred.astype(jnp.int32))


def _body(k_ref, x_ref, o_ref):
    x = x_ref[...]
    xc = jnp.clip(x, 0.0, 1.0)
    s = _rowsum(xc)
    xb = lax.bitcast_convert_type(xc, jnp.int32)

    def vstep(_, carry):
        lo, hi = carry
        mid = (lo + hi) >> 1
        ge = _count(xb >= mid) >= _K
        return jnp.where(ge, mid, lo), jnp.where(ge, hi, mid)

    r = x.shape[0]
    lo0 = jnp.zeros((r, 1), jnp.int32)
    hi0 = jnp.full((r, 1), 0x3F800001, jnp.int32)
    lo, _ = lax.fori_loop(0, 31, vstep, (lo0, hi0))
    t = lo

    eq = xb == t
    n_ge = _count(xb >= t)
    n_gt = _count(xb > t)
    need = _K - n_gt
    idx = lax.broadcasted_iota(jnp.int32, x.shape, 1)

    def tie_bisect():
        def jstep(_, carry):
            jlo, jhi = carry
            mid = (jlo + jhi) >> 1
            geq = _count(eq & (idx < mid)) >= need
            return jnp.where(geq, jlo, mid), jnp.where(geq, mid, jhi)

        jlo0 = jnp.zeros((r, 1), jnp.int32)
        jhi0 = jnp.full((r, 1), _N, jnp.int32)
        _, jhi = lax.fori_loop(0, 16, jstep, (jlo0, jhi0))
        return jhi

    jhi = lax.cond(jnp.all(n_ge == _K),
                   lambda: jnp.full((r, 1), _N, jnp.int32),
                   tie_bisect)

    mask = (xb > t) | (eq & (idx < jhi))
    cond = s > k_ref[0, 0]
    o_ref[...] = jnp.where(cond, jnp.where(mask, xc, 0.0), xc)


def kernel(scores, k):
    rows = scores.shape[0]
    kf = jnp.asarray(k, jnp.float32).reshape(1, 1)
    grid = (rows // _ROWS_PER_BLOCK,)
    return pl.pallas_call(
        _body,
        grid=grid,
        in_specs=[
            pl.BlockSpec(memory_space=pltpu.SMEM),
            pl.BlockSpec((_ROWS_PER_BLOCK, _N), lambda i: (i, 0)),
        ],
        out_specs=pl.BlockSpec((_ROWS_PER_BLOCK, _N), lambda i: (i, 0)),
        out_shape=jax.ShapeDtypeStruct(scores.shape, scores.dtype),
    )(kf, scores)
